# baseline (speedup 1.0000x reference)
import functools

import jax
import jax.numpy as jnp
import numpy as np
from jax import lax
from jax.experimental import pallas as pl
from jax.experimental.pallas import tpu as pltpu
from jax.experimental.pallas import tpu_sc as plsc

NC, NS = 2, 16
NW = NC * NS
BATCH = 2048
NPTS = 128
NSEG = NPTS - 1
HW = 28
NPIX = HW * HW
LANES = 16
STRIDE = LANES + 1
NCHUNK = BATCH // LANES
CPW = NCHUNK // NW
SWORDS = NPTS * 2
CWORDS = SWORDS * LANES
BWORDS = NPIX * LANES
MSHIFT = 20

_EPS = np.float32(1e-7)
_PSET = np.float32(np.float32(1.0) - _EPS)
_A = np.float32(np.log(_EPS))
_B = np.float32(np.log(_PSET))
_A2 = np.float32(np.log(np.float32(np.float32(1.0) - _PSET)))

_MAGIC = np.zeros(32, np.int32)
for _d in range(1, HW):
    _MAGIC[_d] = (2**MSHIFT + _d - 1) // _d


def _sc_body(pc_hbm, tc_hbm, bm_hbm, magic_hbm, out_hbm,
             pcv0, pcv1, tcv0, tcv1, bmv0, bmv1,
             ivp, canvas, outv, magic_v, sem0, sem1):
    wid = lax.axis_index("c") * NS + lax.axis_index("s")

    lane = lax.iota(jnp.int32, LANES)
    lane17 = lane * STRIDE
    zeros = jnp.zeros((LANES,), jnp.float32)
    ones = jnp.ones((LANES,), jnp.float32)

    pltpu.sync_copy(magic_hbm, magic_v)

    def zb(p, _):
        canvas[pl.ds(p * LANES, LANES)] = zeros
        return 0
    lax.fori_loop(0, (NPIX * STRIDE) // LANES, zb, 0)

    bufs = ((pcv0, tcv0, bmv0, sem0), (pcv1, tcv1, bmv1, sem1))

    def issue(j, buf):
        pcv, tcv, bmv, sem = buf
        c = wid * CPW + j
        return (
            pltpu.async_copy(pc_hbm.at[pl.ds(c * CWORDS, CWORDS)], pcv, sem),
            pltpu.async_copy(tc_hbm.at[pl.ds(c * CWORDS, CWORDS)], tcv, sem),
            pltpu.async_copy(bm_hbm.at[pl.ds(c * BWORDS, BWORDS)], bmv, sem),
        )

    n_acc = zeros
    t_acc = zeros
    ta_acc = zeros
    mse_acc = zeros

    pending = issue(0, bufs[0])
    for j in range(CPW):
        pcv, tcv, bmv, sem = bufs[j % 2]
        for h in pending:
            h.wait()
        if j + 1 < CPW:
            pending = issue(j + 1, bufs[(j + 1) % 2])

        def stage_body(g, _):
            for l in range(LANES):
                v = pcv[pl.ds(l * SWORDS + g * LANES, LANES)]
                plsc.store_scatter(ivp, [lane17 + (g * LANES * STRIDE + l)], v)
            return 0
        lax.fori_loop(0, SWORDS // LANES, stage_body, 0)

        def seg_body(k, _):
            o = k * (2 * STRIDE)
            x0f = ivp[pl.ds(o, LANES)]
            y0f = ivp[pl.ds(o + STRIDE, LANES)]
            x1f = ivp[pl.ds(o + 2 * STRIDE, LANES)]
            y1f = ivp[pl.ds(o + 3 * STRIDE, LANES)]
            s = jnp.float32(HW - 1)
            x0 = (x0f * s).astype(jnp.int32)
            y0 = (y0f * s).astype(jnp.int32)
            x1 = (x1f * s).astype(jnp.int32)
            y1 = (y1f * s).astype(jnp.int32)

            steep = jnp.abs(y1 - y0) > jnp.abs(x1 - x0)
            ax0 = jnp.where(steep, y0, x0)
            ay0 = jnp.where(steep, x0, y0)
            ax1 = jnp.where(steep, y1, x1)
            ay1 = jnp.where(steep, x1, y1)
            swap = ax0 > ax1
            bx0 = jnp.where(swap, ax1, ax0)
            bx1 = jnp.where(swap, ax0, ax1)
            by0 = jnp.where(swap, ay1, ay0)
            by1 = jnp.where(swap, ay0, ay1)
            dx = bx1 - bx0
            dy = jnp.abs(by1 - by0)
            den = jnp.maximum(dx, 1)
            up = by0 < by1

            dyM = dy * plsc.load_gather(magic_v, [den])

            rr0 = jnp.where(steep, bx0, by0)
            cc0 = jnp.where(steep, by0, bx0)
            idx0 = (rr0 * HW + cc0) * STRIDE + lane
            step_x = jnp.where(steep, jnp.int32(HW * STRIDE),
                               jnp.int32(STRIDE))
            sy_mag = jnp.where(steep, jnp.int32(STRIDE),
                               jnp.int32(HW * STRIDE))
            step_y = jnp.where(up, sy_mag, -sy_mag)

            plsc.store_scatter(canvas, [idx0], ones)
            xacc = idx0
            for i in range(1, HW):
                xacc = xacc + step_x
                q = (dyM * i) >> MSHIFT
                m = dx >= i
                plsc.store_scatter(canvas, [xacc + q * step_y], ones, mask=m)
            return 0

        lax.fori_loop(0, NSEG, seg_body, 0)

        def red_outer(l, accs):
            def red_body(g, accs):
                na, ta, taa = accs
                cidx = lane17 + (g * LANES * STRIDE + l)
                cv = plsc.load_gather(canvas, [cidx])
                plsc.store_scatter(canvas, [cidx], zeros)
                t = bmv[pl.ds(l * NPIX + g * LANES, LANES)]
                return (na + cv, ta + cv * t, taa + t)
            return lax.fori_loop(0, NPIX // LANES, red_body, accs)

        n_acc, t_acc, ta_acc = lax.fori_loop(
            0, LANES, red_outer, (n_acc, t_acc, ta_acc))

        def mse_body(n, acc):
            for u in range(2):
                o = (n * 2 + u) * LANES
                d = pcv[pl.ds(o, LANES)] - tcv[pl.ds(o, LANES)]
                acc = acc + d * d
            return acc

        mse_acc = lax.fori_loop(0, CWORDS // (2 * LANES), mse_body, mse_acc)

    outv[pl.ds(0, LANES)] = n_acc
    outv[pl.ds(LANES, LANES)] = t_acc
    outv[pl.ds(2 * LANES, LANES)] = ta_acc
    outv[pl.ds(3 * LANES, LANES)] = mse_acc
    pltpu.sync_copy(outv, out_hbm.at[wid])


@functools.partial(jax.jit, static_argnames=())
def kernel(pred_coords, target_coords, target_bitmap):
    pc = pred_coords.reshape(BATCH * NPTS * 2)
    tc = target_coords.reshape(BATCH * NPTS * 2)
    bm = target_bitmap.reshape(BATCH * NPIX)
    magic = jnp.asarray(_MAGIC)

    mesh = plsc.VectorSubcoreMesh(
        core_axis_name="c", subcore_axis_name="s",
        num_cores=NC, num_subcores=NS)

    run = pl.kernel(
        _sc_body,
        out_type=jax.ShapeDtypeStruct((NW, 4 * LANES), jnp.float32),
        mesh=mesh,
        compiler_params=pltpu.CompilerParams(needs_layout_passes=False),
        scratch_types=[
            pltpu.VMEM((CWORDS,), jnp.float32),
            pltpu.VMEM((CWORDS,), jnp.float32),
            pltpu.VMEM((CWORDS,), jnp.float32),
            pltpu.VMEM((CWORDS,), jnp.float32),
            pltpu.VMEM((BWORDS,), jnp.float32),
            pltpu.VMEM((BWORDS,), jnp.float32),
            pltpu.VMEM((SWORDS * STRIDE,), jnp.float32),
            pltpu.VMEM((NPIX * STRIDE,), jnp.float32),
            pltpu.VMEM((4 * LANES,), jnp.float32),
            pltpu.VMEM((32,), jnp.int32),
            pltpu.SemaphoreType.DMA,
            pltpu.SemaphoreType.DMA,
        ],
    )

    parts = run(pc, tc, bm, magic)
    parts = parts.reshape(NW, 4, LANES).sum(axis=(0, 2))
    n_set, t_set, t_all, sse = parts[0], parts[1], parts[2], parts[3]

    n_pix = np.float32(BATCH * NPIX)
    n_coord = np.float32(BATCH * NPTS * 2)
    coord_loss = sse / n_coord
    bce_sum = ((-_B) * n_pix - (_A - _B) * t_all
               + (_B - _A2) * n_set + (_A2 + _A - 2.0 * _B) * t_set)
    raster_loss = bce_sum / n_pix
    total_loss = (np.float32(1.0) * coord_loss
                  + np.float32(0.5) * raster_loss)
    return (coord_loss, raster_loss, total_loss)
